# baseline (speedup 1.0000x reference)
import functools

import jax
import jax.numpy as jnp
from jax import lax
from jax.experimental import pallas as pl
from jax.experimental.pallas import tpu as pltpu
from jax.experimental.pallas import tpu_sc as plsc

_B = 16384
_NC, _NS = 2, 16
_NW = _NC * _NS
_BPW = _B // _NW

_BLK = 512
_NBLK = _B // _BLK


def _make_sc_gather():
  mesh = plsc.VectorSubcoreMesh(
      core_axis_name="c", subcore_axis_name="s",
      num_cores=_NC, num_subcores=_NS)

  nchunk = _BPW // 128

  @functools.partial(
      pl.kernel,
      out_type=jax.ShapeDtypeStruct((16, _B), jnp.float32),
      mesh=mesh,
      compiler_params=pltpu.CompilerParams(
          use_tc_tiling_on_sc=False, needs_layout_passes=False),
      scratch_types=[
          pltpu.VMEM((nchunk, 128), jnp.int32),
          pltpu.VMEM((nchunk, 128), jnp.int32),
          pltpu.VMEM((nchunk, 128), jnp.int32),
          pltpu.VMEM((nchunk, 128), jnp.int32),
          pltpu.VMEM((nchunk, 128), jnp.int32),
          pltpu.VMEM((nchunk, 128), jnp.int32),
          pltpu.VMEM((2, 128, 128), jnp.float32),
          pltpu.VMEM((2, 128, 128), jnp.float32),
          pltpu.VMEM((2, 128, 128), jnp.float32),
          pltpu.VMEM((16, _BPW), jnp.float32),
          pltpu.SemaphoreType.DMA,
          pltpu.SemaphoreType.DMA,
          pltpu.SemaphoreType.DMA,
      ],
  )
  def sc_gather(idx_u_hbm, idx_p_hbm, idx_m_hbm,
                utab_hbm, ptab_hbm, mtab_hbm,
                out_hbm,
                iu_v, ip_v, im_v, du_v, dp_v, dm_v,
                ru_v, rp_v, rm_v, t_v,
                sem_u, sem_p, sem_m):
    wid = lax.axis_index("s") * _NC + lax.axis_index("c")
    base = wid * _BPW
    crow = wid * nchunk
    pltpu.sync_copy(idx_u_hbm.at[pl.ds(crow, nchunk)], iu_v)
    pltpu.sync_copy(idx_p_hbm.at[pl.ds(crow, nchunk)], ip_v)
    pltpu.sync_copy(idx_m_hbm.at[pl.ds(crow, nchunk)], im_v)
    for src, dst in ((iu_v, du_v), (ip_v, dp_v), (im_v, dm_v)):
      for t in range(nchunk):
        for k in range(8):
          chunk = src[t, pl.ds(k * 16, 16)]
          dst[t, pl.ds(k * 16, 16)] = lax.shift_right_logical(chunk, 5)
    tabs = ((iu_v, du_v, ru_v, utab_hbm, sem_u, 0),
            (ip_v, dp_v, rp_v, ptab_hbm, sem_p, 4),
            (im_v, dm_v, rm_v, mtab_hbm, sem_m, 8))
    iota16 = lax.iota(jnp.int32, 16)

    def fire(j, slot):
      cs = []
      for _, dv, rv, tab, sem, _roff in tabs:
        cs.append(pltpu.async_copy(tab.at[dv.at[j]], rv.at[slot], sem))
      return cs

    def extract(j, slot):
      for iv, _dv, rv, _tab, _sem, roff in tabs:
        for g in range(8):
          rows16 = iota16 + (g * 16)
          idx16 = iv[j, pl.ds(g * 16, 16)]
          lane4 = lax.shift_left(jnp.bitwise_and(idx16, 31), 2)
          for c in range(4):
            vals = plsc.load_gather(rv.at[slot], (rows16, lane4 + c))
            t_v[roff + c, pl.ds(j * 128 + g * 16, 16)] = vals

    inflight = fire(0, 0)
    for j in range(nchunk):
      nxt = fire(j + 1, (j + 1) % 2) if j + 1 < nchunk else []
      for c in inflight:
        c.wait()
      extract(j, j % 2)
      inflight = nxt
    for r in range(12):
      pltpu.sync_copy(t_v.at[r], out_hbm.at[r, pl.ds(base, _BPW)])

  return sc_gather


_sc_gather_cache = []


def _get_sc_gather():
  if not _sc_gather_cache:
    _sc_gather_cache.append(_make_sc_gather())
  return _sc_gather_cache[0]


_SMALL = (
    (12, 14, 0, 3),
    (14, 15, 3, 4),
    (15, 18, 14, 5),
    (18, 34, 20, 6),
    (34, 42, 37, 7),
    (42, 50, 46, 8),
)


def _tc_body(x_ref, upm_ref,
             g_ref, a_ref, r_ref, c_ref, s_ref, mt_ref,
             W1_ref, b1_ref, W2_ref, b2_ref, o_ref):
  xb = x_ref[...]
  W1 = W1_ref[...]
  upm = upm_ref[...]
  acc = lax.dot_general(upm[0:12, :], W1[0:12, :],
                        (((0,), (0,)), ((), ())),
                        preferred_element_type=jnp.float32)
  acc += jnp.dot(xb[:, 9:11], W1[50:52], preferred_element_type=jnp.float32)

  small_refs = (g_ref, a_ref, r_ref, c_ref, s_ref, mt_ref)
  lane = lax.broadcasted_iota(jnp.int32, (_BLK, 128), 1)
  oh = jnp.zeros((_BLK, 128), jnp.float32)
  fused_rows = []
  for ref, (lo, hi, off, col) in zip(small_refs, _SMALL):
    tgt = xb[:, col:col + 1].astype(jnp.int32) + off
    oh += (lane == tgt).astype(jnp.float32)
    fused_rows.append(
        jnp.dot(ref[...], W1[lo:hi], preferred_element_type=jnp.float32))
  pad = 128 - sum(f.shape[0] for f in fused_rows)
  cf = jnp.concatenate(fused_rows + [jnp.zeros((pad, 8), jnp.float32)], axis=0)
  acc += jnp.dot(oh, cf, preferred_element_type=jnp.float32)

  h = jnp.maximum(acc + b1_ref[...], 0.0)
  o = jnp.dot(h, W2_ref[...], preferred_element_type=jnp.float32) + b2_ref[...]
  o_ref[...] = jax.nn.sigmoid(o)


def _full(shape):
  return pl.BlockSpec(shape, lambda i: (0,) * len(shape))


_tc_mlp = pl.pallas_call(
    _tc_body,
    grid=(_NBLK,),
    in_specs=[
        pl.BlockSpec((_BLK, 11), lambda i: (i, 0)),
        pl.BlockSpec((16, _BLK), lambda i: (0, i)),
        _full((3, 2)), _full((11, 1)), _full((6, 3)),
        _full((17, 16)), _full((9, 8)), _full((9, 8)),
        _full((52, 8)), _full((1, 8)), _full((8, 1)), _full((1, 1)),
    ],
    out_specs=pl.BlockSpec((_BLK, 1), lambda i: (i, 0)),
    out_shape=jax.ShapeDtypeStruct((_B, 1), jnp.float32),
)


def kernel(x, user_tab, product_tab, model_tab, gender_tab, age_tab,
           res_tab, color_tab, size_tab, mat_tab, W1, b1, W2, b2):
  one = lax.optimization_barrier(jnp.float32(1.0))
  ut = jnp.pad(user_tab, ((0, 31), (0, 0))).reshape(-1, 128) * one
  pt = jnp.pad(product_tab, ((0, 31), (0, 0))).reshape(-1, 128) * one
  mt = jnp.pad(model_tab, ((0, 23), (0, 0))).reshape(-1, 128) * one
  idx_u = x[:, 0].astype(jnp.int32).reshape(_B // 128, 128)
  idx_p = x[:, 1].astype(jnp.int32).reshape(_B // 128, 128)
  idx_m = x[:, 2].astype(jnp.int32).reshape(_B // 128, 128)
  upm = _get_sc_gather()(idx_u, idx_p, idx_m, ut, pt, mt)
  out = _tc_mlp(x, upm, gender_tab, age_tab, res_tab, color_tab,
                size_tab, mat_tab, W1, b1.reshape(1, 8), W2, b2.reshape(1, 1))
  return out[:, 0]


# speedup vs baseline: 1.0008x; 1.0008x over previous
import functools

import jax
import jax.numpy as jnp
from jax import lax
from jax.experimental import pallas as pl
from jax.experimental.pallas import tpu as pltpu
from jax.experimental.pallas import tpu_sc as plsc

_B = 16384
_NC, _NS = 2, 16
_NW = _NC * _NS
_BPW = _B // _NW

_BLK = 512
_NBLK = _B // _BLK


def _make_sc_gather():
  mesh = plsc.VectorSubcoreMesh(
      core_axis_name="c", subcore_axis_name="s",
      num_cores=_NC, num_subcores=_NS)

  nchunk = _BPW // 128

  @functools.partial(
      pl.kernel,
      out_type=jax.ShapeDtypeStruct((16, _B), jnp.float32),
      mesh=mesh,
      compiler_params=pltpu.CompilerParams(needs_layout_passes=False),
      scratch_types=[
          pltpu.VMEM((nchunk, 128), jnp.int32),
          pltpu.VMEM((nchunk, 128), jnp.int32),
          pltpu.VMEM((nchunk, 128), jnp.int32),
          pltpu.VMEM((nchunk, 128), jnp.int32),
          pltpu.VMEM((nchunk, 128), jnp.int32),
          pltpu.VMEM((nchunk, 128), jnp.int32),
          pltpu.VMEM((2, 128, 128), jnp.float32),
          pltpu.VMEM((2, 128, 128), jnp.float32),
          pltpu.VMEM((2, 128, 128), jnp.float32),
          pltpu.VMEM((16, _BPW), jnp.float32),
          pltpu.SemaphoreType.DMA,
          pltpu.SemaphoreType.DMA,
          pltpu.SemaphoreType.DMA,
      ],
  )
  def sc_gather(idx_u_hbm, idx_p_hbm, idx_m_hbm,
                utab_hbm, ptab_hbm, mtab_hbm,
                out_hbm,
                iu_v, ip_v, im_v, du_v, dp_v, dm_v,
                ru_v, rp_v, rm_v, t_v,
                sem_u, sem_p, sem_m):
    wid = lax.axis_index("s") * _NC + lax.axis_index("c")
    base = wid * _BPW
    crow = wid * nchunk
    pltpu.sync_copy(idx_u_hbm.at[pl.ds(crow, nchunk)], iu_v)
    pltpu.sync_copy(idx_p_hbm.at[pl.ds(crow, nchunk)], ip_v)
    pltpu.sync_copy(idx_m_hbm.at[pl.ds(crow, nchunk)], im_v)
    for src, dst in ((iu_v, du_v), (ip_v, dp_v), (im_v, dm_v)):
      for t in range(nchunk):
        for k in range(8):
          chunk = src[t, pl.ds(k * 16, 16)]
          dst[t, pl.ds(k * 16, 16)] = lax.shift_right_logical(chunk, 5)
    tabs = ((iu_v, du_v, ru_v, utab_hbm, sem_u, 0),
            (ip_v, dp_v, rp_v, ptab_hbm, sem_p, 4),
            (im_v, dm_v, rm_v, mtab_hbm, sem_m, 8))
    iota16 = lax.iota(jnp.int32, 16)

    def fire(j, slot):
      cs = []
      for _, dv, rv, tab, sem, _roff in tabs:
        cs.append(pltpu.async_copy(tab.at[dv.at[j]], rv.at[slot], sem))
      return cs

    def extract(j, slot):
      for iv, _dv, rv, _tab, _sem, roff in tabs:
        for g in range(8):
          rows16 = iota16 + (g * 16)
          idx16 = iv[j, pl.ds(g * 16, 16)]
          lane4 = lax.shift_left(jnp.bitwise_and(idx16, 31), 2)
          for c in range(4):
            vals = plsc.load_gather(rv.at[slot], (rows16, lane4 + c))
            t_v[roff + c, pl.ds(j * 128 + g * 16, 16)] = vals

    inflight = fire(0, 0)
    for j in range(nchunk):
      nxt = fire(j + 1, (j + 1) % 2) if j + 1 < nchunk else []
      for c in inflight:
        c.wait()
      extract(j, j % 2)
      inflight = nxt
    for r in range(12):
      pltpu.sync_copy(t_v.at[r], out_hbm.at[r, pl.ds(base, _BPW)])

  return sc_gather


_sc_gather_cache = []


def _get_sc_gather():
  if not _sc_gather_cache:
    _sc_gather_cache.append(_make_sc_gather())
  return _sc_gather_cache[0]


_SMALL = (
    (12, 14, 0, 3),
    (14, 15, 3, 4),
    (15, 18, 14, 5),
    (18, 34, 20, 6),
    (34, 42, 37, 7),
    (42, 50, 46, 8),
)


def _tc_body(x_ref, upm_ref,
             g_ref, a_ref, r_ref, c_ref, s_ref, mt_ref,
             W1_ref, b1_ref, W2_ref, b2_ref, o_ref):
  xb = x_ref[...]
  W1 = W1_ref[...]
  upm = upm_ref[...]
  acc = lax.dot_general(upm[0:12, :], W1[0:12, :],
                        (((0,), (0,)), ((), ())),
                        preferred_element_type=jnp.float32)
  acc += jnp.dot(xb[:, 9:11], W1[50:52], preferred_element_type=jnp.float32)

  small_refs = (g_ref, a_ref, r_ref, c_ref, s_ref, mt_ref)
  lane = lax.broadcasted_iota(jnp.int32, (_BLK, 128), 1)
  oh = jnp.zeros((_BLK, 128), jnp.float32)
  fused_rows = []
  for ref, (lo, hi, off, col) in zip(small_refs, _SMALL):
    tgt = xb[:, col:col + 1].astype(jnp.int32) + off
    oh += (lane == tgt).astype(jnp.float32)
    fused_rows.append(
        jnp.dot(ref[...], W1[lo:hi], preferred_element_type=jnp.float32))
  pad = 128 - sum(f.shape[0] for f in fused_rows)
  cf = jnp.concatenate(fused_rows + [jnp.zeros((pad, 8), jnp.float32)], axis=0)
  acc += jnp.dot(oh, cf, preferred_element_type=jnp.float32)

  h = jnp.maximum(acc + b1_ref[...], 0.0)
  o = jnp.dot(h, W2_ref[...], preferred_element_type=jnp.float32) + b2_ref[...]
  o_ref[...] = jax.nn.sigmoid(o)


def _full(shape):
  return pl.BlockSpec(shape, lambda i: (0,) * len(shape))


_tc_mlp = pl.pallas_call(
    _tc_body,
    grid=(_NBLK,),
    in_specs=[
        pl.BlockSpec((_BLK, 11), lambda i: (i, 0)),
        pl.BlockSpec((16, _BLK), lambda i: (0, i)),
        _full((3, 2)), _full((11, 1)), _full((6, 3)),
        _full((17, 16)), _full((9, 8)), _full((9, 8)),
        _full((52, 8)), _full((1, 8)), _full((8, 1)), _full((1, 1)),
    ],
    out_specs=pl.BlockSpec((_BLK, 1), lambda i: (i, 0)),
    out_shape=jax.ShapeDtypeStruct((_B, 1), jnp.float32),
)


def kernel(x, user_tab, product_tab, model_tab, gender_tab, age_tab,
           res_tab, color_tab, size_tab, mat_tab, W1, b1, W2, b2):
  one = lax.optimization_barrier(jnp.float32(1.0))
  ut = jnp.pad(user_tab, ((0, 31), (0, 0))).reshape(-1, 128) * one
  pt = jnp.pad(product_tab, ((0, 31), (0, 0))).reshape(-1, 128) * one
  mt = jnp.pad(model_tab, ((0, 23), (0, 0))).reshape(-1, 128) * one
  idx_u = x[:, 0].astype(jnp.int32).reshape(_B // 128, 128)
  idx_p = x[:, 1].astype(jnp.int32).reshape(_B // 128, 128)
  idx_m = x[:, 2].astype(jnp.int32).reshape(_B // 128, 128)
  upm = _get_sc_gather()(idx_u, idx_p, idx_m, ut, pt, mt)
  out = _tc_mlp(x, upm, gender_tab, age_tab, res_tab, color_tab,
                size_tab, mat_tab, W1, b1.reshape(1, 8), W2, b2.reshape(1, 1))
  return out[:, 0]


# speedup vs baseline: 1.0018x; 1.0010x over previous
import functools

import jax
import jax.numpy as jnp
from jax import lax
from jax.experimental import pallas as pl
from jax.experimental.pallas import tpu as pltpu
from jax.experimental.pallas import tpu_sc as plsc
from jax.experimental import layout as jax_layout

_B = 16384
_NC, _NS = 2, 16
_NW = _NC * _NS
_BPW = _B // _NW

_BLK = 512
_NBLK = _B // _BLK


def _make_sc_gather():
  mesh = plsc.VectorSubcoreMesh(
      core_axis_name="c", subcore_axis_name="s",
      num_cores=_NC, num_subcores=_NS)

  nchunk = _BPW // 128

  @functools.partial(
      pl.kernel,
      out_type=jax.ShapeDtypeStruct((16, _B), jnp.float32),
      mesh=mesh,
      compiler_params=pltpu.CompilerParams(needs_layout_passes=False),
      scratch_types=[
          pltpu.VMEM((nchunk, 128), jnp.int32),
          pltpu.VMEM((nchunk, 128), jnp.int32),
          pltpu.VMEM((nchunk, 128), jnp.int32),
          pltpu.VMEM((nchunk, 128), jnp.int32),
          pltpu.VMEM((nchunk, 128), jnp.int32),
          pltpu.VMEM((nchunk, 128), jnp.int32),
          pltpu.VMEM((2, 128, 128), jnp.float32),
          pltpu.VMEM((2, 128, 128), jnp.float32),
          pltpu.VMEM((2, 128, 128), jnp.float32),
          pltpu.VMEM((16, _BPW), jnp.float32),
          pltpu.SemaphoreType.DMA,
          pltpu.SemaphoreType.DMA,
          pltpu.SemaphoreType.DMA,
      ],
  )
  def sc_gather(idx_u_hbm, idx_p_hbm, idx_m_hbm,
                utab_hbm, ptab_hbm, mtab_hbm,
                out_hbm,
                iu_v, ip_v, im_v, du_v, dp_v, dm_v,
                ru_v, rp_v, rm_v, t_v,
                sem_u, sem_p, sem_m):
    wid = lax.axis_index("s") * _NC + lax.axis_index("c")
    base = wid * _BPW
    crow = wid * nchunk
    pltpu.sync_copy(idx_u_hbm.at[pl.ds(crow, nchunk)], iu_v)
    pltpu.sync_copy(idx_p_hbm.at[pl.ds(crow, nchunk)], ip_v)
    pltpu.sync_copy(idx_m_hbm.at[pl.ds(crow, nchunk)], im_v)
    for src, dst in ((iu_v, du_v), (ip_v, dp_v), (im_v, dm_v)):
      for t in range(nchunk):
        for k in range(8):
          chunk = src[t, pl.ds(k * 16, 16)]
          dst[t, pl.ds(k * 16, 16)] = lax.shift_right_logical(chunk, 5)
    tabs = ((iu_v, du_v, ru_v, utab_hbm, sem_u, 0),
            (ip_v, dp_v, rp_v, ptab_hbm, sem_p, 4),
            (im_v, dm_v, rm_v, mtab_hbm, sem_m, 8))
    iota16 = lax.iota(jnp.int32, 16)

    def fire(j, slot):
      cs = []
      for _, dv, rv, tab, sem, _roff in tabs:
        cs.append(pltpu.async_copy(tab.at[dv.at[j]], rv.at[slot], sem))
      return cs

    def extract(j, slot):
      for iv, _dv, rv, _tab, _sem, roff in tabs:
        for g in range(8):
          rows16 = iota16 + (g * 16)
          idx16 = iv[j, pl.ds(g * 16, 16)]
          lane4 = lax.shift_left(jnp.bitwise_and(idx16, 31), 2)
          for c in range(4):
            vals = plsc.load_gather(rv.at[slot], (rows16, lane4 + c))
            t_v[roff + c, pl.ds(j * 128 + g * 16, 16)] = vals

    inflight = fire(0, 0)
    for j in range(nchunk):
      nxt = fire(j + 1, (j + 1) % 2) if j + 1 < nchunk else []
      for c in inflight:
        c.wait()
      extract(j, j % 2)
      inflight = nxt
    for r in range(12):
      pltpu.sync_copy(t_v.at[r], out_hbm.at[r, pl.ds(base, _BPW)])

  return sc_gather


_sc_gather_cache = []


def _get_sc_gather():
  if not _sc_gather_cache:
    _sc_gather_cache.append(_make_sc_gather())
  return _sc_gather_cache[0]


_SMALL = (
    (12, 14, 0, 3),
    (14, 15, 3, 4),
    (15, 18, 14, 5),
    (18, 34, 20, 6),
    (34, 42, 37, 7),
    (42, 50, 46, 8),
)


def _tc_body(x_ref, upm_ref,
             g_ref, a_ref, r_ref, c_ref, s_ref, mt_ref,
             W1_ref, b1_ref, W2_ref, b2_ref, o_ref):
  xb = x_ref[...]
  W1 = W1_ref[...]
  upm = upm_ref[...]
  acc = lax.dot_general(upm[0:12, :], W1[0:12, :],
                        (((0,), (0,)), ((), ())),
                        preferred_element_type=jnp.float32)
  acc += jnp.dot(xb[:, 9:11], W1[50:52], preferred_element_type=jnp.float32)

  small_refs = (g_ref, a_ref, r_ref, c_ref, s_ref, mt_ref)
  lane = lax.broadcasted_iota(jnp.int32, (_BLK, 128), 1)
  oh = jnp.zeros((_BLK, 128), jnp.float32)
  fused_rows = []
  for ref, (lo, hi, off, col) in zip(small_refs, _SMALL):
    tgt = xb[:, col:col + 1].astype(jnp.int32) + off
    oh += (lane == tgt).astype(jnp.float32)
    fused_rows.append(
        jnp.dot(ref[...], W1[lo:hi], preferred_element_type=jnp.float32))
  pad = 128 - sum(f.shape[0] for f in fused_rows)
  cf = jnp.concatenate(fused_rows + [jnp.zeros((pad, 8), jnp.float32)], axis=0)
  acc += jnp.dot(oh, cf, preferred_element_type=jnp.float32)

  h = jnp.maximum(acc + b1_ref[...], 0.0)
  o = jnp.dot(h, W2_ref[...], preferred_element_type=jnp.float32) + b2_ref[...]
  o_ref[...] = jax.nn.sigmoid(o)


def _full(shape):
  return pl.BlockSpec(shape, lambda i: (0,) * len(shape))


_tc_mlp = pl.pallas_call(
    _tc_body,
    grid=(_NBLK,),
    in_specs=[
        pl.BlockSpec((_BLK, 11), lambda i: (i, 0)),
        pl.BlockSpec((16, _BLK), lambda i: (0, i)),
        _full((3, 2)), _full((11, 1)), _full((6, 3)),
        _full((17, 16)), _full((9, 8)), _full((9, 8)),
        _full((52, 8)), _full((1, 8)), _full((8, 1)), _full((1, 1)),
    ],
    out_specs=pl.BlockSpec((_BLK, 1), lambda i: (i, 0)),
    out_shape=jax.ShapeDtypeStruct((_B, 1), jnp.float32),
)


def kernel(x, user_tab, product_tab, model_tab, gender_tab, age_tab,
           res_tab, color_tab, size_tab, mat_tab, W1, b1, W2, b2):
  one = lax.optimization_barrier(jnp.float32(1.0))
  rowmajor = jax_layout.Layout(major_to_minor=(0, 1), tiling=((8, 128),))
  sharding = jax.sharding.SingleDeviceSharding(jax.devices()[0])

  def to_rowmajor(v):
    v = jax.lax.with_sharding_constraint(v, sharding)
    return jax_layout.with_layout_constraint(v, rowmajor)

  ut = to_rowmajor(jnp.pad(user_tab, ((0, 31), (0, 0))).reshape(-1, 128) * one)
  pt = to_rowmajor(
      jnp.pad(product_tab, ((0, 31), (0, 0))).reshape(-1, 128) * one)
  mt = to_rowmajor(jnp.pad(model_tab, ((0, 23), (0, 0))).reshape(-1, 128) * one)
  idx_u = x[:, 0].astype(jnp.int32).reshape(_B // 128, 128)
  idx_p = x[:, 1].astype(jnp.int32).reshape(_B // 128, 128)
  idx_m = x[:, 2].astype(jnp.int32).reshape(_B // 128, 128)
  upm = _get_sc_gather()(idx_u, idx_p, idx_m, ut, pt, mt)
  out = _tc_mlp(x, upm, gender_tab, age_tab, res_tab, color_tab,
                size_tab, mat_tab, W1, b1.reshape(1, 8), W2, b2.reshape(1, 1))
  return out[:, 0]


# speedup vs baseline: 8.2164x; 8.2020x over previous
import functools

import jax
import jax.numpy as jnp
from jax import lax
from jax.experimental import pallas as pl
from jax.experimental.pallas import tpu as pltpu
from jax.experimental.pallas import tpu_sc as plsc
from jax.experimental import layout as jax_layout

_B = 16384
_NC, _NS = 2, 16
_NW = _NC * _NS
_BPW = _B // _NW

_BLK = 512
_NBLK = _B // _BLK


def _make_sc_gather():
  mesh = plsc.VectorSubcoreMesh(
      core_axis_name="c", subcore_axis_name="s",
      num_cores=_NC, num_subcores=_NS)

  nchunk = _BPW // 128

  @functools.partial(
      pl.kernel,
      out_type=jax.ShapeDtypeStruct((16, _B), jnp.float32),
      mesh=mesh,
      compiler_params=pltpu.CompilerParams(needs_layout_passes=False),
      scratch_types=[
          pltpu.VMEM((nchunk, 128), jnp.int32),
          pltpu.VMEM((nchunk, 128), jnp.int32),
          pltpu.VMEM((nchunk, 128), jnp.int32),
          pltpu.VMEM((4, 128), jnp.int32),
          pltpu.VMEM((4, 128, 128), jnp.float32),
          pltpu.VMEM((32, 128), jnp.float32),
          pltpu.VMEM((16, _BPW), jnp.float32),
          pltpu.SemaphoreType.DMA,
      ],
  )
  def sc_gather(idx_u_hbm, idx_p_hbm, idx_m_hbm,
                utab_hbm, ptab_hbm, mtab_hbm,
                out_hbm,
                iu_v, ip_v, im_v, didx_v, rows_v, mt_v, t_v,
                sem):
    wid = lax.axis_index("s") * _NC + lax.axis_index("c")
    base = wid * _BPW
    crow = wid * nchunk
    pltpu.sync_copy(idx_u_hbm.at[pl.ds(crow, nchunk)], iu_v)
    pltpu.sync_copy(idx_p_hbm.at[pl.ds(crow, nchunk)], ip_v)
    pltpu.sync_copy(idx_m_hbm.at[pl.ds(crow, nchunk)], im_v)
    pltpu.sync_copy(mtab_hbm, mt_v)
    iota16 = lax.iota(jnp.int32, 16)
    for iv, tab, roff in ((iu_v, utab_hbm, 0), (ip_v, ptab_hbm, 4)):
      for j in range(nchunk):
        for g in range(8):
          idx16 = iv[j, pl.ds(g * 16, 16)]
          base16 = lax.shift_left(lax.shift_right_logical(idx16, 7), 2)
          for c in range(4):
            didx_v[c, pl.ds(g * 16, 16)] = base16 + c
        cs = [pltpu.async_copy(tab.at[didx_v.at[c]], rows_v.at[c], sem)
              for c in range(4)]
        for cc in cs:
          cc.wait()
        for g in range(8):
          rows16 = iota16 + (g * 16)
          idx16 = iv[j, pl.ds(g * 16, 16)]
          lane16 = jnp.bitwise_and(idx16, 127)
          for c in range(4):
            vals = plsc.load_gather(rows_v.at[c], (rows16, lane16))
            t_v[roff + c, pl.ds(j * 128 + g * 16, 16)] = vals
    for j in range(nchunk):
      for g in range(8):
        idx16 = im_v[j, pl.ds(g * 16, 16)]
        trow16 = lax.shift_left(lax.shift_right_logical(idx16, 7), 2)
        lane16 = jnp.bitwise_and(idx16, 127)
        for c in range(4):
          vals = plsc.load_gather(mt_v, (trow16 + c, lane16))
          t_v[8 + c, pl.ds(j * 128 + g * 16, 16)] = vals
    for r in range(12):
      pltpu.sync_copy(t_v.at[r], out_hbm.at[r, pl.ds(base, _BPW)])

  return sc_gather


_sc_gather_cache = []


def _get_sc_gather():
  if not _sc_gather_cache:
    _sc_gather_cache.append(_make_sc_gather())
  return _sc_gather_cache[0]


_SMALL = (
    (12, 14, 0, 3),
    (14, 15, 3, 4),
    (15, 18, 14, 5),
    (18, 34, 20, 6),
    (34, 42, 37, 7),
    (42, 50, 46, 8),
)


def _tc_body(x_ref, upm_ref,
             g_ref, a_ref, r_ref, c_ref, s_ref, mt_ref,
             W1_ref, b1_ref, W2_ref, b2_ref, o_ref):
  xb = x_ref[...]
  W1 = W1_ref[...]
  upm = upm_ref[...]
  acc = lax.dot_general(upm[0:12, :], W1[0:12, :],
                        (((0,), (0,)), ((), ())),
                        preferred_element_type=jnp.float32)
  acc += jnp.dot(xb[:, 9:11], W1[50:52], preferred_element_type=jnp.float32)

  small_refs = (g_ref, a_ref, r_ref, c_ref, s_ref, mt_ref)
  lane = lax.broadcasted_iota(jnp.int32, (_BLK, 128), 1)
  oh = jnp.zeros((_BLK, 128), jnp.float32)
  fused_rows = []
  for ref, (lo, hi, off, col) in zip(small_refs, _SMALL):
    tgt = xb[:, col:col + 1].astype(jnp.int32) + off
    oh += (lane == tgt).astype(jnp.float32)
    fused_rows.append(
        jnp.dot(ref[...], W1[lo:hi], preferred_element_type=jnp.float32))
  pad = 128 - sum(f.shape[0] for f in fused_rows)
  cf = jnp.concatenate(fused_rows + [jnp.zeros((pad, 8), jnp.float32)], axis=0)
  acc += jnp.dot(oh, cf, preferred_element_type=jnp.float32)

  h = jnp.maximum(acc + b1_ref[...], 0.0)
  o = jnp.dot(h, W2_ref[...], preferred_element_type=jnp.float32) + b2_ref[...]
  o_ref[...] = jax.nn.sigmoid(o)


def _full(shape):
  return pl.BlockSpec(shape, lambda i: (0,) * len(shape))


_tc_mlp = pl.pallas_call(
    _tc_body,
    grid=(_NBLK,),
    in_specs=[
        pl.BlockSpec((_BLK, 11), lambda i: (i, 0)),
        pl.BlockSpec((16, _BLK), lambda i: (0, i)),
        _full((3, 2)), _full((11, 1)), _full((6, 3)),
        _full((17, 16)), _full((9, 8)), _full((9, 8)),
        _full((52, 8)), _full((1, 8)), _full((8, 1)), _full((1, 1)),
    ],
    out_specs=pl.BlockSpec((_BLK, 1), lambda i: (i, 0)),
    out_shape=jax.ShapeDtypeStruct((_B, 1), jnp.float32),
)


def kernel(x, user_tab, product_tab, model_tab, gender_tab, age_tab,
           res_tab, color_tab, size_tab, mat_tab, W1, b1, W2, b2):
  def native_view(tab, pad_rows):
    tp = jnp.pad(tab, ((0, pad_rows), (0, 0)))
    nt = tp.shape[0] // 128
    return tp.reshape(nt, 128, 4).transpose(0, 2, 1).reshape(nt * 4, 128)

  ut = native_view(user_tab, 63)
  pt = native_view(product_tab, 95)
  mt = native_view(model_tab, 23)
  idx_u = x[:, 0].astype(jnp.int32).reshape(_B // 128, 128)
  idx_p = x[:, 1].astype(jnp.int32).reshape(_B // 128, 128)
  idx_m = x[:, 2].astype(jnp.int32).reshape(_B // 128, 128)
  upm = _get_sc_gather()(idx_u, idx_p, idx_m, ut, pt, mt)
  out = _tc_mlp(x, upm, gender_tab, age_tab, res_tab, color_tab,
                size_tab, mat_tab, W1, b1.reshape(1, 8), W2, b2.reshape(1, 1))
  return out[:, 0]
